# speedup vs baseline: 2.4504x; 2.4504x over previous
import functools

import jax
import jax.numpy as jnp
from jax.experimental import pallas as pl
from jax.experimental.pallas import tpu as pltpu

_B, _N, _D = 8, 1024, 256


def _normalize_body(f_ref, out_ref):
    f = f_ref[0]
    n2 = jnp.sum(f * f, axis=-1, keepdims=True)
    norm = jnp.maximum(jnp.sqrt(n2), 1e-8)
    out_ref[0] = (f / norm).astype(jnp.bfloat16)


def _normalize(features):
    return pl.pallas_call(
        _normalize_body,
        grid=(_B,),
        in_specs=[pl.BlockSpec((1, _N, _D), lambda b: (b, 0, 0))],
        out_specs=pl.BlockSpec((1, _N, _D), lambda b: (b, 0, 0)),
        out_shape=jax.ShapeDtypeStruct((_B, _N, _D), jnp.bfloat16),
    )(features)


def _loss_body(pts_ref, pptsT_ref, fn_ref, fnT_ref, tri_ref, out_ref):
    i = pl.program_id(0)
    j = pl.program_id(1)

    a = pts_ref[i]
    pT = pptsT_ref[i, j]

    a2 = jnp.sum(a * a, axis=-1, keepdims=True)
    b2 = jnp.sum(pT * pT, axis=0, keepdims=True)
    ab = jax.lax.dot(a, pT, preferred_element_type=jnp.float32)
    d2 = jnp.maximum(a2 + b2 - 2.0 * ab, 0.0)
    mask = (d2 <= 1.0) & tri_ref[...]

    c = jax.lax.dot(fn_ref[j], fnT_ref[i],
                    preferred_element_type=jnp.float32)
    s = jnp.sum(jnp.where(mask, 1.0 - c, 0.0))

    @pl.when((i == 0) & (j == 0))
    def _():
        out_ref[0, 0] = 0.0

    out_ref[0, 0] += s


def _match_loss(pts, pptsT, fn, fnT, tri):
    out = pl.pallas_call(
        _loss_body,
        grid=(_B, _B),
        in_specs=[
            pl.BlockSpec((_B, _N, 2), lambda i, j: (0, 0, 0)),
            pl.BlockSpec((_B, _B, 2, _N), lambda i, j: (0, 0, 0, 0)),
            pl.BlockSpec((_B, _N, _D), lambda i, j: (0, 0, 0)),
            pl.BlockSpec((_B, _D, _N), lambda i, j: (0, 0, 0)),
            pl.BlockSpec((_N, _N), lambda i, j: (0, 0)),
        ],
        out_specs=pl.BlockSpec(memory_space=pltpu.SMEM),
        out_shape=jax.ShapeDtypeStruct((1, 1), jnp.float32),
    )(pts, pptsT, fn, fnT, tri)
    return out[0, 0]


def kernel(features, points, proj_pts, invis_idx, height, width):
    B, N, _ = points.shape
    factor = jnp.array([(width - 1.0) / 2.0, (height - 1.0) / 2.0],
                       dtype=points.dtype)
    pts = (points + 1.0) * factor
    ppts = (proj_pts + 1.0) * factor
    pptsT = jnp.swapaxes(ppts, 2, 3)

    fn = _normalize(features)
    fnT = jnp.swapaxes(fn, 1, 2)

    tri = jnp.triu(jnp.ones((N, N), dtype=jnp.bool_))
    return _match_loss(pts, pptsT, fn, fnT, tri)


# speedup vs baseline: 4.4357x; 1.8102x over previous
import functools

import jax
import jax.numpy as jnp
from jax.experimental import pallas as pl
from jax.experimental.pallas import tpu as pltpu

_B, _N, _D = 8, 1024, 256


def _normalize_body(f_ref, out_ref):
    f = f_ref[0]
    n2 = jnp.sum(f * f, axis=-1, keepdims=True)
    norm = jnp.maximum(jnp.sqrt(n2), 1e-8)
    out_ref[0] = (f / norm).astype(jnp.bfloat16)


def _normalize(features):
    return pl.pallas_call(
        _normalize_body,
        grid=(_B,),
        in_specs=[pl.BlockSpec((1, _N, _D), lambda b: (b, 0, 0))],
        out_specs=pl.BlockSpec((1, _N, _D), lambda b: (b, 0, 0)),
        out_shape=jax.ShapeDtypeStruct((_B, _N, _D), jnp.bfloat16),
    )(features)


_T = 256


def _loss_body(pts_ref, pptsT_ref, fn_ref, fnT_ref, tri_ref, out_ref):
    i = pl.program_id(0)
    j = pl.program_id(1)

    acc = jnp.float32(0.0)
    for tn in range(_N // _T):
        a = pts_ref[i, pl.ds(tn * _T, _T)]
        a2 = jnp.sum(a * a, axis=-1, keepdims=True)
        fj = fn_ref[j, pl.ds(tn * _T, _T)]
        for tm in range(tn, _N // _T):
            pT = pptsT_ref[i, j, :, pl.ds(tm * _T, _T)]
            b2 = jnp.sum(pT * pT, axis=0, keepdims=True)
            ab = jax.lax.dot(a, pT, preferred_element_type=jnp.float32)
            d2 = a2 + b2 - 2.0 * ab
            mask = d2 <= 1.0
            if tn == tm:
                mask = mask & tri_ref[...]
            fiT = fnT_ref[i, :, pl.ds(tm * _T, _T)]
            c = jax.lax.dot(fj, fiT, preferred_element_type=jnp.float32)
            acc += jnp.sum(jnp.where(mask, 1.0 - c, 0.0))

    @pl.when((i == 0) & (j == 0))
    def _():
        out_ref[0, 0] = 0.0

    out_ref[0, 0] += acc


def _match_loss(pts, pptsT, fn, fnT, tri):
    out = pl.pallas_call(
        _loss_body,
        grid=(_B, _B),
        in_specs=[
            pl.BlockSpec((_B, _N, 2), lambda i, j: (0, 0, 0)),
            pl.BlockSpec((_B, _B, 2, _N), lambda i, j: (0, 0, 0, 0)),
            pl.BlockSpec((_B, _N, _D), lambda i, j: (0, 0, 0)),
            pl.BlockSpec((_B, _D, _N), lambda i, j: (0, 0, 0)),
            pl.BlockSpec((_T, _T), lambda i, j: (0, 0)),
        ],
        out_specs=pl.BlockSpec(memory_space=pltpu.SMEM),
        out_shape=jax.ShapeDtypeStruct((1, 1), jnp.float32),
    )(pts, pptsT, fn, fnT, tri)
    return out[0, 0]


def kernel(features, points, proj_pts, invis_idx, height, width):
    B, N, _ = points.shape
    factor = jnp.array([(width - 1.0) / 2.0, (height - 1.0) / 2.0],
                       dtype=points.dtype)
    pts = (points + 1.0) * factor
    ppts = (proj_pts + 1.0) * factor
    pptsT = jnp.swapaxes(ppts, 2, 3)

    fn = _normalize(features)
    fnT = jnp.swapaxes(fn, 1, 2)

    tri = jnp.triu(jnp.ones((_T, _T), dtype=jnp.bool_))
    return _match_loss(pts, pptsT, fn, fnT, tri)


# speedup vs baseline: 4.6985x; 1.0593x over previous
import functools

import jax
import jax.numpy as jnp
from jax.experimental import pallas as pl
from jax.experimental.pallas import tpu as pltpu

_B, _N, _D = 8, 1024, 256


def _normalize_body(f_ref, out_ref):
    f = f_ref[0]
    n2 = jnp.sum(f * f, axis=-1, keepdims=True)
    norm = jnp.maximum(jnp.sqrt(n2), 1e-8)
    out_ref[0] = (f / norm).astype(jnp.bfloat16)


def _normalize(features):
    return pl.pallas_call(
        _normalize_body,
        grid=(_B,),
        in_specs=[pl.BlockSpec((1, _N, _D), lambda b: (b, 0, 0))],
        out_specs=pl.BlockSpec((1, _N, _D), lambda b: (b, 0, 0)),
        out_shape=jax.ShapeDtypeStruct((_B, _N, _D), jnp.bfloat16),
    )(features)


_T = 256
_DN_T = (((1,), (1,)), ((), ()))


def _loss_body(pts_ref, pptsT_ref, fn_ref, tri_ref, out_ref):
    i = pl.program_id(0)
    j = pl.program_id(1)

    acc = jnp.float32(0.0)
    for tn in range(_N // _T):
        a = pts_ref[i, pl.ds(tn * _T, _T)]
        a2 = jnp.sum(a * a, axis=-1, keepdims=True)
        fj = fn_ref[j, pl.ds(tn * _T, _T)]
        for tm in range(tn, _N // _T):
            pT = pptsT_ref[i, j, :, pl.ds(tm * _T, _T)]
            b2 = jnp.sum(pT * pT, axis=0, keepdims=True)
            ab = jax.lax.dot(a, pT, preferred_element_type=jnp.float32)
            d2 = a2 + b2 - 2.0 * ab
            mask = d2 <= 1.0
            if tn == tm:
                mask = mask & tri_ref[...]
            fi = fn_ref[i, pl.ds(tm * _T, _T)]
            c = jax.lax.dot_general(fj, fi, _DN_T,
                                    preferred_element_type=jnp.float32)
            acc += jnp.sum(jnp.where(mask, 1.0 - c, 0.0))

    @pl.when((i == 0) & (j == 0))
    def _():
        out_ref[0, 0] = 0.0

    out_ref[0, 0] += acc


def _match_loss(pts, pptsT, fn, tri):
    out = pl.pallas_call(
        _loss_body,
        grid=(_B, _B),
        in_specs=[
            pl.BlockSpec((_B, _N, 2), lambda i, j: (0, 0, 0)),
            pl.BlockSpec((_B, _B, 2, _N), lambda i, j: (0, 0, 0, 0)),
            pl.BlockSpec((_B, _N, _D), lambda i, j: (0, 0, 0)),
            pl.BlockSpec((_T, _T), lambda i, j: (0, 0)),
        ],
        out_specs=pl.BlockSpec(memory_space=pltpu.SMEM),
        out_shape=jax.ShapeDtypeStruct((1, 1), jnp.float32),
    )(pts, pptsT, fn, tri)
    return out[0, 0]


def kernel(features, points, proj_pts, invis_idx, height, width):
    B, N, _ = points.shape
    factor = jnp.array([(width - 1.0) / 2.0, (height - 1.0) / 2.0],
                       dtype=points.dtype)
    pts = (points + 1.0) * factor
    ppts = (proj_pts + 1.0) * factor
    pptsT = jnp.swapaxes(ppts, 2, 3)

    fn = _normalize(features)

    tri = jnp.triu(jnp.ones((_T, _T), dtype=jnp.bool_))
    return _match_loss(pts, pptsT, fn, tri)


# speedup vs baseline: 5.0381x; 1.0723x over previous
import jax
import jax.numpy as jnp
from jax.experimental import pallas as pl
from jax.experimental.pallas import tpu as pltpu

_B, _N, _D = 8, 1024, 256
_T = 256
_DN_T = (((1,), (1,)), ((), ()))


def _loss_body(feat_ref, pts_ref, pptsT_ref, tri_ref, out_ref, fn_scr):
    i = pl.program_id(0)
    j = pl.program_id(1)

    @pl.when((i == 0) & (j == 0))
    def _():
        out_ref[0, 0] = 0.0
        for b in range(_B):
            f = feat_ref[b]
            n2 = jnp.sum(f * f, axis=-1, keepdims=True)
            norm = jnp.maximum(jnp.sqrt(n2), 1e-8)
            fn_scr[b] = (f / norm).astype(jnp.bfloat16)

    acc = jnp.float32(0.0)
    for tn in range(_N // _T):
        a = pts_ref[i, pl.ds(tn * _T, _T)]
        a2 = jnp.sum(a * a, axis=-1, keepdims=True)
        fj = fn_scr[j, pl.ds(tn * _T, _T)]
        for tm in range(tn, _N // _T):
            pT = pptsT_ref[i, j, :, pl.ds(tm * _T, _T)]
            b2 = jnp.sum(pT * pT, axis=0, keepdims=True)
            ab = jax.lax.dot(a, pT, preferred_element_type=jnp.float32)
            d2 = a2 + b2 - 2.0 * ab
            mask = d2 <= 1.0
            if tn == tm:
                mask = mask & tri_ref[...]
            fi = fn_scr[i, pl.ds(tm * _T, _T)]
            c = jax.lax.dot_general(fj, fi, _DN_T,
                                    preferred_element_type=jnp.float32)
            acc += jnp.sum(jnp.where(mask, 1.0 - c, 0.0))

    out_ref[0, 0] += acc


def _match_loss(features, pts, pptsT, tri):
    out = pl.pallas_call(
        _loss_body,
        grid=(_B, _B),
        in_specs=[
            pl.BlockSpec((_B, _N, _D), lambda i, j: (0, 0, 0)),
            pl.BlockSpec((_B, _N, 2), lambda i, j: (0, 0, 0)),
            pl.BlockSpec((_B, _B, 2, _N), lambda i, j: (0, 0, 0, 0)),
            pl.BlockSpec((_T, _T), lambda i, j: (0, 0)),
        ],
        out_specs=pl.BlockSpec(memory_space=pltpu.SMEM),
        out_shape=jax.ShapeDtypeStruct((1, 1), jnp.float32),
        scratch_shapes=[pltpu.VMEM((_B, _N, _D), jnp.bfloat16)],
    )(features, pts, pptsT, tri)
    return out[0, 0]


def kernel(features, points, proj_pts, invis_idx, height, width):
    factor = jnp.array([(width - 1.0) / 2.0, (height - 1.0) / 2.0],
                       dtype=points.dtype)
    pts = (points + 1.0) * factor
    ppts = (proj_pts + 1.0) * factor
    pptsT = jnp.swapaxes(ppts, 2, 3)

    tri = jnp.triu(jnp.ones((_T, _T), dtype=jnp.bool_))
    return _match_loss(features, pts, pptsT, tri)
